# baseline (speedup 1.0000x reference)
import functools

import jax
import jax.numpy as jnp
from jax import lax
from jax.experimental import pallas as pl
from jax.experimental.pallas import tpu as pltpu
from jax.experimental.pallas import tpu_sc as plsc

_FOV = 5.0
_SBIN = 25
_C_KMS = 299792.458
_H0 = 70.0
_OMEGA_M = 0.3

_N = 5_000_000
_CHUNK = 10000
_NCHUNKS = _N // _CHUNK
_NWORKERS = 32
_VECS = _CHUNK // 16


def _bin_params(z):
    zs = jnp.linspace(0.0, 1.0, 257) * z
    inv_ez = 1.0 / jnp.sqrt(_OMEGA_M * (1.0 + zs) ** 3 + (1.0 - _OMEGA_M))
    dz = zs[1:] - zs[:-1]
    integ = jnp.sum(0.5 * (inv_ez[1:] + inv_ez[:-1]) * dz)
    d_c = (_C_KMS / _H0) * integ
    d_a = d_c / (1.0 + z)
    ang_kpc_per_arcsec = d_a * 1000.0 * (jnp.pi / (180.0 * 3600.0))
    aperture = _FOV * ang_kpc_per_arcsec
    lo = -aperture / 2.0
    inv_step = _SBIN / aperture
    return lo, inv_step


def _sc_body(params_hbm, x_hbm, y_hbm, out_hbm, pbuf, xbuf, ybuf, obuf):
    wid = lax.axis_index("s") * 2 + lax.axis_index("c")
    pltpu.sync_copy(params_hbm, pbuf)
    scale = pbuf[pl.ds(0, 16)]
    offs = pbuf[pl.ds(16, 16)]

    nch = jnp.where(wid < _NCHUNKS - 32 * (_NCHUNKS // 32), 1, 0) + _NCHUNKS // 32

    def chunk_body(t, _):
        c = wid + 32 * t
        base = c * _CHUNK
        pltpu.sync_copy(x_hbm.at[pl.ds(base, _CHUNK)], xbuf)
        pltpu.sync_copy(y_hbm.at[pl.ds(base, _CHUNK)], ybuf)

        def vec_body(v, _):
            x = xbuf[pl.ds(v * 16, 16)]
            y = ybuf[pl.ds(v * 16, 16)]
            ix = jnp.clip((x * scale + offs).astype(jnp.int32), 0, _SBIN - 1)
            iy = jnp.clip((y * scale + offs).astype(jnp.int32), 0, _SBIN - 1)
            obuf[pl.ds(v * 16, 16)] = (ix + _SBIN * iy).astype(jnp.float32)
            return 0

        lax.fori_loop(0, _VECS, vec_body, 0, unroll=5)
        pltpu.sync_copy(obuf, out_hbm.at[pl.ds(base, _CHUNK)])
        return 0

    lax.fori_loop(0, nch, chunk_body, 0)


def kernel(coords, galaxy_dist_z):
    z = jnp.squeeze(galaxy_dist_z)
    lo, inv_step = _bin_params(z)
    params = jnp.concatenate([
        jnp.full((16,), inv_step, dtype=jnp.float32),
        jnp.full((16,), -lo * inv_step, dtype=jnp.float32),
    ])
    x = coords[:, 0]
    y = coords[:, 1]

    mesh = plsc.VectorSubcoreMesh(core_axis_name="c", subcore_axis_name="s")
    run = functools.partial(
        pl.kernel,
        mesh=mesh,
        compiler_params=pltpu.CompilerParams(use_tc_tiling_on_sc=True),
        out_type=jax.ShapeDtypeStruct((_N,), jnp.float32),
        scratch_types=[
            pltpu.VMEM((32,), jnp.float32),
            pltpu.VMEM((_CHUNK,), jnp.float32),
            pltpu.VMEM((_CHUNK,), jnp.float32),
            pltpu.VMEM((_CHUNK,), jnp.float32),
        ],
    )(_sc_body)
    return run(params, x, y)


# speedup vs baseline: 1.3650x; 1.3650x over previous
import functools

import jax
import jax.numpy as jnp
from jax import lax
from jax.experimental import pallas as pl
from jax.experimental.pallas import tpu as pltpu
from jax.experimental.pallas import tpu_sc as plsc

_FOV = 5.0
_SBIN = 25
_C_KMS = 299792.458
_H0 = 70.0
_OMEGA_M = 0.3

_N = 5_000_000
_CHUNK = 10000
_NCHUNKS = _N // _CHUNK
_NWORKERS = 32
_VECS = _CHUNK // 16


def _bin_params(z):
    zs = jnp.linspace(0.0, 1.0, 257) * z
    inv_ez = 1.0 / jnp.sqrt(_OMEGA_M * (1.0 + zs) ** 3 + (1.0 - _OMEGA_M))
    dz = zs[1:] - zs[:-1]
    integ = jnp.sum(0.5 * (inv_ez[1:] + inv_ez[:-1]) * dz)
    d_c = (_C_KMS / _H0) * integ
    d_a = d_c / (1.0 + z)
    ang_kpc_per_arcsec = d_a * 1000.0 * (jnp.pi / (180.0 * 3600.0))
    aperture = _FOV * ang_kpc_per_arcsec
    lo = -aperture / 2.0
    inv_step = _SBIN / aperture
    return lo, inv_step


def _sc_body(params_hbm, x_hbm, y_hbm, out_hbm, pbuf, xbuf, ybuf, obuf,
             xsem, ysem):
    wid = lax.axis_index("s") * 2 + lax.axis_index("c")
    pltpu.sync_copy(params_hbm, pbuf)
    scale = pbuf[pl.ds(0, 16)]
    offs = pbuf[pl.ds(16, 16)]

    nch = jnp.where(wid < _NCHUNKS - 32 * (_NCHUNKS // 32), 1, 0) + _NCHUNKS // 32

    def chunk_body(t, _):
        c = wid + 32 * t
        base = c * _CHUNK
        hx = pltpu.async_copy(x_hbm.at[pl.ds(base, _CHUNK)], xbuf, xsem)
        hy = pltpu.async_copy(y_hbm.at[pl.ds(base, _CHUNK)], ybuf, ysem)
        hx.wait()
        hy.wait()

        @plsc.parallel_loop(0, _CHUNK, step=16, unroll=5)
        def vec_body(i):
            x = xbuf[pl.ds(i, 16)]
            y = ybuf[pl.ds(i, 16)]
            ix = jnp.clip((x * scale + offs).astype(jnp.int32), 0, _SBIN - 1)
            iy = jnp.clip((y * scale + offs).astype(jnp.int32), 0, _SBIN - 1)
            obuf[pl.ds(i, 16)] = (ix + _SBIN * iy).astype(jnp.float32)

        pltpu.sync_copy(obuf, out_hbm.at[pl.ds(base, _CHUNK)])
        return 0

    lax.fori_loop(0, nch, chunk_body, 0)


def kernel(coords, galaxy_dist_z):
    z = jnp.squeeze(galaxy_dist_z)
    lo, inv_step = _bin_params(z)
    params = jnp.concatenate([
        jnp.full((16,), inv_step, dtype=jnp.float32),
        jnp.full((16,), -lo * inv_step, dtype=jnp.float32),
    ])
    x = coords[:, 0]
    y = coords[:, 1]

    mesh = plsc.VectorSubcoreMesh(core_axis_name="c", subcore_axis_name="s")
    run = functools.partial(
        pl.kernel,
        mesh=mesh,
        compiler_params=pltpu.CompilerParams(use_tc_tiling_on_sc=True),
        out_type=jax.ShapeDtypeStruct((_N,), jnp.float32),
        scratch_types=[
            pltpu.VMEM((32,), jnp.float32),
            pltpu.VMEM((_CHUNK,), jnp.float32),
            pltpu.VMEM((_CHUNK,), jnp.float32),
            pltpu.VMEM((_CHUNK,), jnp.float32),
            pltpu.SemaphoreType.DMA,
            pltpu.SemaphoreType.DMA,
        ],
    )(_sc_body)
    return run(params, x, y)


# speedup vs baseline: 1.4860x; 1.0886x over previous
import functools

import jax
import jax.numpy as jnp
from jax import lax
from jax.experimental import pallas as pl
from jax.experimental.pallas import tpu as pltpu
from jax.experimental.pallas import tpu_sc as plsc

_FOV = 5.0
_SBIN = 25
_C_KMS = 299792.458
_H0 = 70.0
_OMEGA_M = 0.3

_N = 5_000_000
_CHUNK = 10000
_NCHUNKS = _N // _CHUNK
_NWORKERS = 32
_VECS = _CHUNK // 16


def _bin_params(z):
    zs = jnp.linspace(0.0, 1.0, 257) * z
    inv_ez = 1.0 / jnp.sqrt(_OMEGA_M * (1.0 + zs) ** 3 + (1.0 - _OMEGA_M))
    dz = zs[1:] - zs[:-1]
    integ = jnp.sum(0.5 * (inv_ez[1:] + inv_ez[:-1]) * dz)
    d_c = (_C_KMS / _H0) * integ
    d_a = d_c / (1.0 + z)
    ang_kpc_per_arcsec = d_a * 1000.0 * (jnp.pi / (180.0 * 3600.0))
    aperture = _FOV * ang_kpc_per_arcsec
    lo = -aperture / 2.0
    inv_step = _SBIN / aperture
    return lo, inv_step


def _sc_body(params_hbm, x_hbm, y_hbm, out_hbm, pbuf,
             xb0, yb0, ob0, xb1, yb1, ob1,
             xs0, ys0, os0, xs1, ys1, os1):
    wid = lax.axis_index("s") * 2 + lax.axis_index("c")
    pltpu.sync_copy(params_hbm, pbuf)
    scale = pbuf[pl.ds(0, 16)]
    offs = pbuf[pl.ds(16, 16)]

    nch = jnp.where(wid < _NCHUNKS - 32 * (_NCHUNKS // 32), 1, 0) + _NCHUNKS // 32

    def cbase(t):
        return (wid + 32 * t) * _CHUNK

    def start_in(t, xb, yb, xs, ys):
        pltpu.async_copy(x_hbm.at[pl.ds(cbase(t), _CHUNK)], xb, xs)
        pltpu.async_copy(y_hbm.at[pl.ds(cbase(t), _CHUNK)], yb, ys)

    def wait_in(t, xb, yb, xs, ys):
        pltpu.make_async_copy(x_hbm.at[pl.ds(cbase(t), _CHUNK)], xb, xs).wait()
        pltpu.make_async_copy(y_hbm.at[pl.ds(cbase(t), _CHUNK)], yb, ys).wait()

    def compute(xb, yb, ob):
        @plsc.parallel_loop(0, _CHUNK, step=16, unroll=5)
        def vec_body(i):
            x = xb[pl.ds(i, 16)]
            y = yb[pl.ds(i, 16)]
            ix = jnp.clip((x * scale + offs).astype(jnp.int32), 0, _SBIN - 1)
            iy = jnp.clip((y * scale + offs).astype(jnp.int32), 0, _SBIN - 1)
            ob[pl.ds(i, 16)] = (ix + _SBIN * iy).astype(jnp.float32)

    def start_out(t, ob, osem):
        pltpu.async_copy(ob, out_hbm.at[pl.ds(cbase(t), _CHUNK)], osem)

    def wait_out(t, ob, osem):
        pltpu.make_async_copy(ob, out_hbm.at[pl.ds(cbase(t), _CHUNK)], osem).wait()

    start_in(0, xb0, yb0, xs0, ys0)
    npairs = (nch + 1) // 2

    def pair_body(p, _):
        t0 = 2 * p
        t1 = t0 + 1

        @pl.when(t1 < nch)
        def _():
            start_in(t1, xb1, yb1, xs1, ys1)

        wait_in(t0, xb0, yb0, xs0, ys0)

        @pl.when(p > 0)
        def _():
            wait_out(2 * (p - 1), ob0, os0)

        compute(xb0, yb0, ob0)
        start_out(t0, ob0, os0)

        @pl.when(t0 + 2 < nch)
        def _():
            start_in(t0 + 2, xb0, yb0, xs0, ys0)

        @pl.when(t1 < nch)
        def _():
            wait_in(t1, xb1, yb1, xs1, ys1)

            @pl.when(p > 0)
            def _():
                wait_out(2 * (p - 1) + 1, ob1, os1)

            compute(xb1, yb1, ob1)
            start_out(t1, ob1, os1)

        return 0

    lax.fori_loop(0, npairs, pair_body, 0)

    wait_out(2 * (npairs - 1), ob0, os0)
    wait_out((nch // 2) * 2 - 1, ob1, os1)


def kernel(coords, galaxy_dist_z):
    z = jnp.squeeze(galaxy_dist_z)
    lo, inv_step = _bin_params(z)
    params = jnp.concatenate([
        jnp.full((16,), inv_step, dtype=jnp.float32),
        jnp.full((16,), -lo * inv_step, dtype=jnp.float32),
    ])
    x = coords[:, 0]
    y = coords[:, 1]

    mesh = plsc.VectorSubcoreMesh(core_axis_name="c", subcore_axis_name="s")
    run = functools.partial(
        pl.kernel,
        mesh=mesh,
        compiler_params=pltpu.CompilerParams(use_tc_tiling_on_sc=True),
        out_type=jax.ShapeDtypeStruct((_N,), jnp.float32),
        scratch_types=[
            pltpu.VMEM((32,), jnp.float32),
            pltpu.VMEM((_CHUNK,), jnp.float32),
            pltpu.VMEM((_CHUNK,), jnp.float32),
            pltpu.VMEM((_CHUNK,), jnp.float32),
            pltpu.VMEM((_CHUNK,), jnp.float32),
            pltpu.VMEM((_CHUNK,), jnp.float32),
            pltpu.VMEM((_CHUNK,), jnp.float32),
            pltpu.SemaphoreType.DMA,
            pltpu.SemaphoreType.DMA,
            pltpu.SemaphoreType.DMA,
            pltpu.SemaphoreType.DMA,
            pltpu.SemaphoreType.DMA,
            pltpu.SemaphoreType.DMA,
        ],
    )(_sc_body)
    return run(params, x, y)


# speedup vs baseline: 1.5035x; 1.0117x over previous
import functools

import jax
import jax.numpy as jnp
from jax import lax
from jax.experimental import pallas as pl
from jax.experimental.pallas import tpu as pltpu
from jax.experimental.pallas import tpu_sc as plsc

_FOV = 5.0
_SBIN = 25
_C_KMS = 299792.458
_H0 = 70.0
_OMEGA_M = 0.3
_ARCSEC_RAD = 3.141592653589793 / (180.0 * 3600.0)

_N = 5_000_000
_CHUNK = 10000
_NCHUNKS = _N // _CHUNK
_NWORKERS = 32


def _dg(src, idx):
    return lax.gather(
        src, idx[:, None],
        lax.GatherDimensionNumbers(
            offset_dims=(), collapsed_slice_dims=(0,), start_index_map=(0,)),
        slice_sizes=(1,), mode=lax.GatherScatterMode.PROMISE_IN_BOUNDS)


def _inv_ez(zi):
    t = 1.0 + zi
    u = _OMEGA_M * (t * t * t) + (1.0 - _OMEGA_M)
    w = jnp.full((16,), 1.0, dtype=jnp.float32)
    for _ in range(4):
        w = 0.5 * (w + u / w)
    return 1.0 / w


def _sc_body(z_hbm, x_hbm, y_hbm, out_hbm, zbuf,
             xb0, yb0, ob0, xb1, yb1, ob1,
             xs0, ys0, os0, xs1, ys1, os1):
    wid = lax.axis_index("s") * 2 + lax.axis_index("c")
    pltpu.sync_copy(z_hbm, zbuf)
    zv = zbuf[pl.ds(0, 16)]
    it = lax.iota(jnp.int32, 16)

    zstep = zv * (1.0 / 256.0)

    def integ_body(j, acc):
        i = j * 16 + it
        zi = i.astype(jnp.float32) * zstep
        f = _inv_ez(zi)
        return acc + jnp.where(i <= 256, f, 0.0)

    acc = lax.fori_loop(0, 17, integ_body, jnp.zeros((16,), jnp.float32))
    for sh in (8, 4, 2, 1):
        acc = acc + _dg(acc, (it + sh) & 15)
    total = acc - 0.5 - 0.5 * _inv_ez(zv)
    integ = total * zstep
    d_a = (_C_KMS / _H0) * integ / (1.0 + zv)
    aperture = _FOV * d_a * 1000.0 * _ARCSEC_RAD
    scale = _SBIN / aperture

    nch = jnp.where(wid < _NCHUNKS - 32 * (_NCHUNKS // 32), 1, 0) + _NCHUNKS // 32

    def cbase(t):
        return (wid + 32 * t) * _CHUNK

    def start_in(t, xb, yb, xs, ys):
        pltpu.async_copy(x_hbm.at[pl.ds(cbase(t), _CHUNK)], xb, xs)
        pltpu.async_copy(y_hbm.at[pl.ds(cbase(t), _CHUNK)], yb, ys)

    def wait_in(t, xb, yb, xs, ys):
        pltpu.make_async_copy(x_hbm.at[pl.ds(cbase(t), _CHUNK)], xb, xs).wait()
        pltpu.make_async_copy(y_hbm.at[pl.ds(cbase(t), _CHUNK)], yb, ys).wait()

    def compute(xb, yb, ob):
        @plsc.parallel_loop(0, _CHUNK, step=16, unroll=5)
        def vec_body(i):
            x = xb[pl.ds(i, 16)]
            y = yb[pl.ds(i, 16)]
            fx = jnp.minimum(jnp.maximum(x * scale + 12.5, 0.0), 24.5)
            fy = jnp.minimum(jnp.maximum(y * scale + 12.5, 0.0), 24.5)
            ix = fx.astype(jnp.int32)
            iy = fy.astype(jnp.int32)
            ob[pl.ds(i, 16)] = (iy * _SBIN + ix).astype(jnp.float32)

    def start_out(t, ob, osem):
        pltpu.async_copy(ob, out_hbm.at[pl.ds(cbase(t), _CHUNK)], osem)

    def wait_out(t, ob, osem):
        pltpu.make_async_copy(ob, out_hbm.at[pl.ds(cbase(t), _CHUNK)], osem).wait()

    start_in(0, xb0, yb0, xs0, ys0)
    npairs = (nch + 1) // 2

    def pair_body(p, _):
        t0 = 2 * p
        t1 = t0 + 1

        @pl.when(t1 < nch)
        def _():
            start_in(t1, xb1, yb1, xs1, ys1)

        wait_in(t0, xb0, yb0, xs0, ys0)

        @pl.when(p > 0)
        def _():
            wait_out(2 * (p - 1), ob0, os0)

        compute(xb0, yb0, ob0)
        start_out(t0, ob0, os0)

        @pl.when(t0 + 2 < nch)
        def _():
            start_in(t0 + 2, xb0, yb0, xs0, ys0)

        @pl.when(t1 < nch)
        def _():
            wait_in(t1, xb1, yb1, xs1, ys1)

            @pl.when(p > 0)
            def _():
                wait_out(2 * (p - 1) + 1, ob1, os1)

            compute(xb1, yb1, ob1)
            start_out(t1, ob1, os1)

        return 0

    lax.fori_loop(0, npairs, pair_body, 0)

    wait_out(2 * (npairs - 1), ob0, os0)
    wait_out((nch // 2) * 2 - 1, ob1, os1)


def kernel(coords, galaxy_dist_z):
    zvec = jnp.broadcast_to(jnp.squeeze(galaxy_dist_z), (16,)).astype(jnp.float32)
    x = coords[:, 0]
    y = coords[:, 1]

    mesh = plsc.VectorSubcoreMesh(core_axis_name="c", subcore_axis_name="s")
    run = functools.partial(
        pl.kernel,
        mesh=mesh,
        compiler_params=pltpu.CompilerParams(use_tc_tiling_on_sc=True),
        out_type=jax.ShapeDtypeStruct((_N,), jnp.float32),
        scratch_types=[
            pltpu.VMEM((16,), jnp.float32),
            pltpu.VMEM((_CHUNK,), jnp.float32),
            pltpu.VMEM((_CHUNK,), jnp.float32),
            pltpu.VMEM((_CHUNK,), jnp.float32),
            pltpu.VMEM((_CHUNK,), jnp.float32),
            pltpu.VMEM((_CHUNK,), jnp.float32),
            pltpu.VMEM((_CHUNK,), jnp.float32),
            pltpu.SemaphoreType.DMA,
            pltpu.SemaphoreType.DMA,
            pltpu.SemaphoreType.DMA,
            pltpu.SemaphoreType.DMA,
            pltpu.SemaphoreType.DMA,
            pltpu.SemaphoreType.DMA,
        ],
    )(_sc_body)
    return run(zvec, x, y)


# speedup vs baseline: 1.5130x; 1.0063x over previous
import functools

import jax
import jax.numpy as jnp
from jax import lax
from jax.experimental import pallas as pl
from jax.experimental.pallas import tpu as pltpu
from jax.experimental.pallas import tpu_sc as plsc

_FOV = 5.0
_SBIN = 25
_C_KMS = 299792.458
_H0 = 70.0
_OMEGA_M = 0.3
_ARCSEC_RAD = 3.141592653589793 / (180.0 * 3600.0)

_N = 5_000_000
_CHUNK = 10000
_NCHUNKS = _N // _CHUNK
_NWORKERS = 32


def _dg(src, idx):
    return lax.gather(
        src, idx[:, None],
        lax.GatherDimensionNumbers(
            offset_dims=(), collapsed_slice_dims=(0,), start_index_map=(0,)),
        slice_sizes=(1,), mode=lax.GatherScatterMode.PROMISE_IN_BOUNDS)


def _inv_ez(zi):
    t = 1.0 + zi
    u = _OMEGA_M * (t * t * t) + (1.0 - _OMEGA_M)
    w = jnp.full((16,), 1.0, dtype=jnp.float32)
    for _ in range(4):
        w = 0.5 * (w + u / w)
    return 1.0 / w


def _sc_body(z_hbm, x_hbm, y_hbm, out_hbm, zbuf,
             xb0, yb0, ob0, xb1, yb1, ob1,
             xs0, ys0, os0, xs1, ys1, os1):
    wid = lax.axis_index("s") * 2 + lax.axis_index("c")
    pltpu.sync_copy(z_hbm, zbuf)
    zv = zbuf[pl.ds(0, 16)]
    it = lax.iota(jnp.int32, 16)

    zstep = zv * (1.0 / 256.0)

    def integ_body(j, acc):
        i = j * 16 + it
        zi = i.astype(jnp.float32) * zstep
        f = _inv_ez(zi)
        return acc + jnp.where(i <= 256, f, 0.0)

    acc = lax.fori_loop(0, 17, integ_body, jnp.zeros((16,), jnp.float32))
    for sh in (8, 4, 2, 1):
        acc = acc + _dg(acc, (it + sh) & 15)
    total = acc - 0.5 - 0.5 * _inv_ez(zv)
    integ = total * zstep
    d_a = (_C_KMS / _H0) * integ / (1.0 + zv)
    aperture = _FOV * d_a * 1000.0 * _ARCSEC_RAD
    scale = _SBIN / aperture

    nch = jnp.where(wid < _NCHUNKS - 32 * (_NCHUNKS // 32), 1, 0) + _NCHUNKS // 32

    def cbase(t):
        return (wid + 32 * t) * _CHUNK

    def start_in(t, xb, yb, xs, ys):
        pltpu.async_copy(x_hbm.at[pl.ds(cbase(t), _CHUNK)], xb, xs)
        pltpu.async_copy(y_hbm.at[pl.ds(cbase(t), _CHUNK)], yb, ys)

    def wait_in(t, xb, yb, xs, ys):
        pltpu.make_async_copy(x_hbm.at[pl.ds(cbase(t), _CHUNK)], xb, xs).wait()
        pltpu.make_async_copy(y_hbm.at[pl.ds(cbase(t), _CHUNK)], yb, ys).wait()

    def compute(xb, yb, ob):
        @plsc.parallel_loop(0, _CHUNK, step=16, unroll=25)
        def vec_body(i):
            x = xb[pl.ds(i, 16)]
            y = yb[pl.ds(i, 16)]
            fx = jnp.minimum(jnp.maximum(x * scale + 12.5, 0.0), 24.5)
            fy = jnp.minimum(jnp.maximum(y * scale + 12.5, 0.0), 24.5)
            ix = fx.astype(jnp.int32)
            iy = fy.astype(jnp.int32)
            ob[pl.ds(i, 16)] = (iy * _SBIN + ix).astype(jnp.float32)

    def start_out(t, ob, osem):
        pltpu.async_copy(ob, out_hbm.at[pl.ds(cbase(t), _CHUNK)], osem)

    def wait_out(t, ob, osem):
        pltpu.make_async_copy(ob, out_hbm.at[pl.ds(cbase(t), _CHUNK)], osem).wait()

    start_in(0, xb0, yb0, xs0, ys0)
    npairs = (nch + 1) // 2

    def pair_body(p, _):
        t0 = 2 * p
        t1 = t0 + 1

        @pl.when(t1 < nch)
        def _():
            start_in(t1, xb1, yb1, xs1, ys1)

        wait_in(t0, xb0, yb0, xs0, ys0)

        @pl.when(p > 0)
        def _():
            wait_out(2 * (p - 1), ob0, os0)

        compute(xb0, yb0, ob0)
        start_out(t0, ob0, os0)

        @pl.when(t0 + 2 < nch)
        def _():
            start_in(t0 + 2, xb0, yb0, xs0, ys0)

        @pl.when(t1 < nch)
        def _():
            wait_in(t1, xb1, yb1, xs1, ys1)

            @pl.when(p > 0)
            def _():
                wait_out(2 * (p - 1) + 1, ob1, os1)

            compute(xb1, yb1, ob1)
            start_out(t1, ob1, os1)

        return 0

    lax.fori_loop(0, npairs, pair_body, 0)

    wait_out(2 * (npairs - 1), ob0, os0)
    wait_out((nch // 2) * 2 - 1, ob1, os1)


def kernel(coords, galaxy_dist_z):
    zvec = jnp.broadcast_to(jnp.squeeze(galaxy_dist_z), (16,)).astype(jnp.float32)
    x = coords[:, 0]
    y = coords[:, 1]

    mesh = plsc.VectorSubcoreMesh(core_axis_name="c", subcore_axis_name="s")
    run = functools.partial(
        pl.kernel,
        mesh=mesh,
        compiler_params=pltpu.CompilerParams(use_tc_tiling_on_sc=True),
        out_type=jax.ShapeDtypeStruct((_N,), jnp.float32),
        scratch_types=[
            pltpu.VMEM((16,), jnp.float32),
            pltpu.VMEM((_CHUNK,), jnp.float32),
            pltpu.VMEM((_CHUNK,), jnp.float32),
            pltpu.VMEM((_CHUNK,), jnp.float32),
            pltpu.VMEM((_CHUNK,), jnp.float32),
            pltpu.VMEM((_CHUNK,), jnp.float32),
            pltpu.VMEM((_CHUNK,), jnp.float32),
            pltpu.SemaphoreType.DMA,
            pltpu.SemaphoreType.DMA,
            pltpu.SemaphoreType.DMA,
            pltpu.SemaphoreType.DMA,
            pltpu.SemaphoreType.DMA,
            pltpu.SemaphoreType.DMA,
        ],
    )(_sc_body)
    return run(zvec, x, y)
